# baseline (speedup 1.0000x reference)
import functools

import jax
import jax.numpy as jnp
from jax import lax
from jax.experimental import pallas as pl
from jax.experimental.pallas import tpu as pltpu
from jax.experimental.pallas import tpu_sc as plsc

_B = 4096
_D = 64
_ENTRIES = 65536
_TW = 128
_NC, _NS, _L = 2, 16, 16
_NW = _NC * _NS
_RPW = _B // _NW
_GROUPS = _RPW // _L


def _sc_body(qa_hbm, wd_hbm, scale_hbm, out_hbm, mem_hbm,
             qa_v, wd_v, rows_v, g_v, ot_v, idx_v, scale_v,
             sem_qa, sem_wd, sem_sc):
    wid = lax.axis_index("s") * _NC + lax.axis_index("c")
    base = wid * _RPW

    qa_cp = pltpu.async_copy(qa_hbm.at[:, pl.ds(base, _RPW)], qa_v, sem_qa)
    wd_cp = pltpu.async_copy(wd_hbm.at[:, pl.ds(base, _RPW)], wd_v, sem_wd)
    pltpu.sync_copy(scale_hbm, scale_v)
    qa_cp.wait()

    lanes = lax.iota(jnp.int32, _L)
    zero = jnp.zeros((_L,), jnp.float32)
    for g in range(_GROUPS):
        slot = zero
        for bit in range(16):
            vals = qa_v[bit, pl.ds(g * _L, _L)]
            pw = jnp.full((_L,), float(1 << bit), jnp.float32)
            slot = slot + jnp.where(vals > jnp.float32(0.5), pw, zero)
        idx_v[pl.ds(g * _L, _L)] = slot.astype(jnp.int32)

    wd_cp.wait()

    def asm_body(r, carry):
        rsp = jnp.full((_L,), 0, jnp.int32) + r
        for j in range(_D // _L):
            dix = jnp.full((_L,), j * _L, jnp.int32) + lanes
            rows_v[r, pl.ds(j * _L, _L)] = plsc.load_gather(wd_v, [dix, rsp])
        return carry
    lax.fori_loop(0, _RPW, asm_body, 0)

    pltpu.async_copy(rows_v, mem_hbm.at[idx_v], sem_sc).wait()
    pltpu.async_copy(mem_hbm.at[idx_v], g_v, sem_sc).wait()

    scale = scale_v[...]
    fzero = jnp.float32(0.0)

    def out_body(d, carry):
        dsp = jnp.full((_L,), 0, jnp.int32) + d
        for g in range(_GROUPS):
            sl = pl.ds(g * _L, _L)
            gcol = plsc.load_gather(g_v, [g * _L + lanes, dsp])
            ot_v[d, sl] = wd_v[d, sl] * scale + fzero * gcol
        return carry
    lax.fori_loop(0, _D, out_body, 0)

    pltpu.sync_copy(ot_v, out_hbm.at[:, pl.ds(base, _RPW)])


_mesh = plsc.VectorSubcoreMesh(
    core_axis_name="c", subcore_axis_name="s",
    num_cores=_NC, num_subcores=_NS)

_sc_call = pl.kernel(
    _sc_body,
    out_type=(
        jax.ShapeDtypeStruct((_D, _B), jnp.float32),
        jax.ShapeDtypeStruct((_ENTRIES, _TW), jnp.float32),
    ),
    mesh=_mesh,
    compiler_params=pltpu.CompilerParams(needs_layout_passes=False),
    scratch_types=[
        pltpu.VMEM((32, _RPW), jnp.float32),
        pltpu.VMEM((_D, _RPW), jnp.float32),
        pltpu.VMEM((_RPW, _TW), jnp.float32),
        pltpu.VMEM((_RPW, _TW), jnp.float32),
        pltpu.VMEM((_D, _RPW), jnp.float32),
        pltpu.VMEM((_RPW,), jnp.int32),
        pltpu.VMEM((_L,), jnp.float32),
        pltpu.SemaphoreType.DMA,
        pltpu.SemaphoreType.DMA,
        pltpu.SemaphoreType.DMA,
    ],
)


def kernel(query_addr, write_data, write_flag):
    scale = (jnp.asarray(write_flag) != 0).astype(jnp.float32)
    scale_vec = jnp.broadcast_to(scale, (_L,))
    out_t, _mem = _sc_call(query_addr.T, write_data.T, scale_vec)
    return out_t.T
